# baseline (speedup 1.0000x reference)
import jax
import jax.numpy as jnp
from jax.experimental import pallas as pl
from jax.experimental.pallas import tpu as pltpu


def _fused_kernel(x_ref, w_ref, b_ref, adj_ref, out_ref, h_ref):
    @pl.when(pl.program_id(0) == 0)
    def _():
        h = jax.lax.dot_general(
            x_ref[...], w_ref[...],
            dimension_numbers=(((1,), (1,)), ((), ())),
            preferred_element_type=jnp.float32,
        )
        h_ref[...] = (h + b_ref[...]).astype(jnp.float32)

    p = jax.lax.dot_general(
        adj_ref[...], h_ref[...],
        dimension_numbers=(((1,), (0,)), ((), ())),
        precision=jax.lax.Precision('bfloat16'),
        preferred_element_type=jnp.float32,
    )
    out_ref[...] = p


@jax.jit
def kernel(x, adj, W, b):
    V, C = x.shape
    O = W.shape[0]
    b2 = b.reshape(1, O)

    BM = 400
    grid = (V // BM,)
    out = pl.pallas_call(
        _fused_kernel,
        grid=grid,
        in_specs=[
            pl.BlockSpec((V, C), lambda m: (0, 0)),
            pl.BlockSpec((O, C), lambda m: (0, 0)),
            pl.BlockSpec((1, O), lambda m: (0, 0)),
            pl.BlockSpec((BM, V), lambda m: (m, 0)),
        ],
        out_specs=pl.BlockSpec((BM, O), lambda m: (m, 0)),
        out_shape=jax.ShapeDtypeStruct((V, O), jnp.float32),
        scratch_shapes=[pltpu.VMEM((V, O), jnp.float32)],
        compiler_params=pltpu.CompilerParams(
            dimension_semantics=("arbitrary",),
        ),
    )(x, W, b2, adj)
    return out
